# baseline (speedup 1.0000x reference)
import jax
import jax.numpy as jnp
from jax.experimental import pallas as pl
from jax.experimental.pallas import tpu as pltpu

BM = 200


def _gcn_kernel(seq_ref, w_ref, adj_ref, out_ref, seq_raw_ref):
    @pl.when(pl.program_id(0) == 0)
    def _():
        seq_raw_ref[...] = jnp.dot(
            seq_ref[...], w_ref[...].T, preferred_element_type=jnp.float32
        ).astype(jnp.bfloat16)

    acc = jnp.dot(adj_ref[...].astype(jnp.bfloat16), seq_raw_ref[...],
                  preferred_element_type=jnp.float32)
    out_ref[...] = jnp.maximum(acc, 0.0)


@jax.jit
def kernel(seq, adj, W):
    n, d_in = seq.shape
    d_out = W.shape[0]
    return pl.pallas_call(
        _gcn_kernel,
        grid=(pl.cdiv(n, BM),),
        in_specs=[
            pl.BlockSpec((n, d_in), lambda i: (0, 0)),
            pl.BlockSpec((d_out, d_in), lambda i: (0, 0)),
            pl.BlockSpec((BM, n), lambda i: (i, 0)),
        ],
        out_specs=pl.BlockSpec((BM, d_out), lambda i: (i, 0)),
        out_shape=jax.ShapeDtypeStruct((n, d_out), jnp.float32),
        scratch_shapes=[pltpu.VMEM((n, d_out), jnp.bfloat16)],
    )(seq, W, adj)


# speedup vs baseline: 1.0148x; 1.0148x over previous
import jax
import jax.numpy as jnp
from jax.experimental import pallas as pl
from jax.experimental.pallas import tpu as pltpu

BM = 256


def _gcn_kernel(seq_ref, w_ref, adj_ref, out_ref, seq_raw_ref):
    @pl.when(pl.program_id(0) == 0)
    def _():
        seq_raw_ref[...] = jnp.dot(
            seq_ref[...], w_ref[...].T, preferred_element_type=jnp.float32
        ).astype(jnp.bfloat16)

    acc = jnp.dot(adj_ref[...].astype(jnp.bfloat16), seq_raw_ref[...],
                  preferred_element_type=jnp.float32)
    out_ref[...] = jnp.maximum(acc, 0.0)


@jax.jit
def kernel(seq, adj, W):
    n, d_in = seq.shape
    d_out = W.shape[0]
    return pl.pallas_call(
        _gcn_kernel,
        grid=(pl.cdiv(n, BM),),
        in_specs=[
            pl.BlockSpec((n, d_in), lambda i: (0, 0)),
            pl.BlockSpec((d_out, d_in), lambda i: (0, 0)),
            pl.BlockSpec((BM, n), lambda i: (i, 0)),
        ],
        out_specs=pl.BlockSpec((BM, d_out), lambda i: (i, 0)),
        out_shape=jax.ShapeDtypeStruct((n, d_out), jnp.float32),
        scratch_shapes=[pltpu.VMEM((n, d_out), jnp.bfloat16)],
    )(seq, W, adj)


# speedup vs baseline: 1.0187x; 1.0039x over previous
import jax
import jax.numpy as jnp
from jax.experimental import pallas as pl
from jax.experimental.pallas import tpu as pltpu

BM = 320


def _gcn_kernel(seq_ref, w_ref, adj_ref, out_ref, seq_raw_ref):
    @pl.when(pl.program_id(0) == 0)
    def _():
        seq_raw_ref[...] = jnp.dot(
            seq_ref[...], w_ref[...].T, preferred_element_type=jnp.float32
        ).astype(jnp.bfloat16)

    acc = jnp.dot(adj_ref[...].astype(jnp.bfloat16), seq_raw_ref[...],
                  preferred_element_type=jnp.float32)
    out_ref[...] = jnp.maximum(acc, 0.0)


@jax.jit
def kernel(seq, adj, W):
    n, d_in = seq.shape
    d_out = W.shape[0]
    return pl.pallas_call(
        _gcn_kernel,
        grid=(pl.cdiv(n, BM),),
        in_specs=[
            pl.BlockSpec((n, d_in), lambda i: (0, 0)),
            pl.BlockSpec((d_out, d_in), lambda i: (0, 0)),
            pl.BlockSpec((BM, n), lambda i: (i, 0)),
        ],
        out_specs=pl.BlockSpec((BM, d_out), lambda i: (i, 0)),
        out_shape=jax.ShapeDtypeStruct((n, d_out), jnp.float32),
        scratch_shapes=[pltpu.VMEM((n, d_out), jnp.bfloat16)],
    )(seq, W, adj)


# speedup vs baseline: 1.0242x; 1.0054x over previous
import jax
import jax.numpy as jnp
from jax.experimental import pallas as pl
from jax.experimental.pallas import tpu as pltpu

BM = 288


def _gcn_kernel(seq_ref, w_ref, adj_ref, out_ref, seq_raw_ref):
    @pl.when(pl.program_id(0) == 0)
    def _():
        seq_raw_ref[...] = jnp.dot(
            seq_ref[...], w_ref[...].T, preferred_element_type=jnp.float32
        ).astype(jnp.bfloat16)

    acc = jnp.dot(adj_ref[...].astype(jnp.bfloat16), seq_raw_ref[...],
                  preferred_element_type=jnp.float32)
    out_ref[...] = jnp.maximum(acc, 0.0)


@jax.jit
def kernel(seq, adj, W):
    n, d_in = seq.shape
    d_out = W.shape[0]
    return pl.pallas_call(
        _gcn_kernel,
        grid=(pl.cdiv(n, BM),),
        in_specs=[
            pl.BlockSpec((n, d_in), lambda i: (0, 0)),
            pl.BlockSpec((d_out, d_in), lambda i: (0, 0)),
            pl.BlockSpec((BM, n), lambda i: (i, 0)),
        ],
        out_specs=pl.BlockSpec((BM, d_out), lambda i: (i, 0)),
        out_shape=jax.ShapeDtypeStruct((n, d_out), jnp.float32),
        scratch_shapes=[pltpu.VMEM((n, d_out), jnp.bfloat16)],
    )(seq, W, adj)


# speedup vs baseline: 1.0252x; 1.0009x over previous
import jax
import jax.numpy as jnp
from jax.experimental import pallas as pl
from jax.experimental.pallas import tpu as pltpu

BM = 272


def _gcn_kernel(seq_ref, w_ref, adj_ref, out_ref, seq_raw_ref):
    @pl.when(pl.program_id(0) == 0)
    def _():
        seq_raw_ref[...] = jnp.dot(
            seq_ref[...], w_ref[...].T, preferred_element_type=jnp.float32
        ).astype(jnp.bfloat16)

    acc = jnp.dot(adj_ref[...].astype(jnp.bfloat16), seq_raw_ref[...],
                  preferred_element_type=jnp.float32)
    out_ref[...] = jnp.maximum(acc, 0.0)


@jax.jit
def kernel(seq, adj, W):
    n, d_in = seq.shape
    d_out = W.shape[0]
    return pl.pallas_call(
        _gcn_kernel,
        grid=(pl.cdiv(n, BM),),
        in_specs=[
            pl.BlockSpec((n, d_in), lambda i: (0, 0)),
            pl.BlockSpec((d_out, d_in), lambda i: (0, 0)),
            pl.BlockSpec((BM, n), lambda i: (i, 0)),
        ],
        out_specs=pl.BlockSpec((BM, d_out), lambda i: (i, 0)),
        out_shape=jax.ShapeDtypeStruct((n, d_out), jnp.float32),
        scratch_shapes=[pltpu.VMEM((n, d_out), jnp.bfloat16)],
    )(seq, W, adj)


# speedup vs baseline: 1.0270x; 1.0018x over previous
import jax
import jax.numpy as jnp
from jax.experimental import pallas as pl
from jax.experimental.pallas import tpu as pltpu

BM = 280


def _gcn_kernel(seq_ref, w_ref, adj_ref, out_ref, seq_raw_ref):
    @pl.when(pl.program_id(0) == 0)
    def _():
        seq_raw_ref[...] = jnp.dot(
            seq_ref[...], w_ref[...].T, preferred_element_type=jnp.float32
        ).astype(jnp.bfloat16)

    acc = jnp.dot(adj_ref[...].astype(jnp.bfloat16), seq_raw_ref[...],
                  preferred_element_type=jnp.float32)
    out_ref[...] = jnp.maximum(acc, 0.0)


@jax.jit
def kernel(seq, adj, W):
    n, d_in = seq.shape
    d_out = W.shape[0]
    return pl.pallas_call(
        _gcn_kernel,
        grid=(pl.cdiv(n, BM),),
        in_specs=[
            pl.BlockSpec((n, d_in), lambda i: (0, 0)),
            pl.BlockSpec((d_out, d_in), lambda i: (0, 0)),
            pl.BlockSpec((BM, n), lambda i: (i, 0)),
        ],
        out_specs=pl.BlockSpec((BM, d_out), lambda i: (i, 0)),
        out_shape=jax.ShapeDtypeStruct((n, d_out), jnp.float32),
        scratch_shapes=[pltpu.VMEM((n, d_out), jnp.bfloat16)],
    )(seq, W, adj)


# speedup vs baseline: 1.0317x; 1.0045x over previous
import jax
import jax.numpy as jnp
from jax.experimental import pallas as pl
from jax.experimental.pallas import tpu as pltpu

BM = 280


def _gcn_kernel(seq_ref, w_ref, adj_ref, out_ref, seq_raw_ref):
    @pl.when(pl.program_id(0) == 0)
    def _():
        seq_raw_ref[...] = jnp.dot(
            seq_ref[...], w_ref[...].T, preferred_element_type=jnp.float32
        ).astype(jnp.bfloat16)

    acc = jax.lax.dot_general(
        adj_ref[...], seq_raw_ref[...],
        (((1,), (0,)), ((), ())),
        preferred_element_type=jnp.float32,
    )
    out_ref[...] = jnp.maximum(acc, 0.0)


@jax.jit
def kernel(seq, adj, W):
    n, d_in = seq.shape
    d_out = W.shape[0]
    return pl.pallas_call(
        _gcn_kernel,
        grid=(pl.cdiv(n, BM),),
        in_specs=[
            pl.BlockSpec((n, d_in), lambda i: (0, 0)),
            pl.BlockSpec((d_out, d_in), lambda i: (0, 0)),
            pl.BlockSpec((BM, n), lambda i: (i, 0)),
        ],
        out_specs=pl.BlockSpec((BM, d_out), lambda i: (i, 0)),
        out_shape=jax.ShapeDtypeStruct((n, d_out), jnp.float32),
        scratch_shapes=[pltpu.VMEM((n, d_out), jnp.bfloat16)],
    )(seq, W, adj)
